# baseline (speedup 1.0000x reference)
import functools

import jax
import jax.numpy as jnp
from jax import lax
from jax.experimental import pallas as pl
from jax.experimental.pallas import tpu as pltpu
from jax.experimental.pallas import tpu_sc as plsc

_IDX_CHUNK = 128


@functools.lru_cache(maxsize=None)
def _make_gather(V2, B):
    info = plsc.get_sparse_core_info()
    NC, NS = info.num_cores, info.num_subcores
    NW = NC * NS
    b_per_w = B // NW
    n_chunks = b_per_w // _IDX_CHUNK
    mesh = plsc.VectorSubcoreMesh(core_axis_name="c", subcore_axis_name="s")

    @functools.partial(
        pl.kernel,
        mesh=mesh,
        out_type=jax.ShapeDtypeStruct((NW, b_per_w, 128), jnp.float32),
        scratch_types=[
            pltpu.VMEM((n_chunks, _IDX_CHUNK), jnp.int32),
            pltpu.VMEM((b_per_w, 128), jnp.float32),
            pltpu.SemaphoreType.DMA,
        ],
        compiler_params=pltpu.CompilerParams(
            use_tc_tiling_on_sc=True, needs_layout_passes=False
        ),
    )
    def gather_k(table_hbm, idx_hbm, out_hbm, idx_v, rows_v, sem):
        wid = lax.axis_index("s") * NC + lax.axis_index("c")
        base = wid * b_per_w
        for j in range(n_chunks):
            pltpu.sync_copy(
                idx_hbm.at[pl.ds(base + j * _IDX_CHUNK, _IDX_CHUNK)],
                idx_v.at[j],
            )
        copies = []
        for j in range(n_chunks):
            copies.append(
                pltpu.async_copy(
                    table_hbm.at[idx_v.at[j]],
                    rows_v.at[pl.ds(j * _IDX_CHUNK, _IDX_CHUNK)],
                    sem,
                )
            )
        for c in copies:
            c.wait()
        pltpu.sync_copy(rows_v, out_hbm.at[wid])

    return gather_k


def _mlp_body(x_ref, par_ref, w1_ref, b1_ref, w2_ref, b2_ref, o_ref):
    x2 = x_ref[...]
    take_hi = par_ref[...] > 0.5
    x = jnp.where(take_hi, x2[:, 64:], x2[:, :64]).astype(jnp.bfloat16)
    h_t = lax.dot_general(
        w1_ref[...].astype(jnp.bfloat16), x, (((1,), (1,)), ((), ())),
        preferred_element_type=jnp.float32,
    )
    h_t = jnp.maximum(h_t + b1_ref[...], 0.0)
    y_t = lax.dot_general(
        w2_ref[...].astype(jnp.bfloat16), h_t.astype(jnp.bfloat16),
        (((1,), (0,)), ((), ())),
        preferred_element_type=jnp.float32,
    )
    y_t = y_t + b2_ref[...]
    norm = jnp.sqrt(jnp.sum(y_t * y_t, axis=0, keepdims=True))
    o_ref[...] = y_t / jnp.maximum(norm, 1e-12)


@functools.lru_cache(maxsize=None)
def _make_mlp(B, D, H, blk):
    grid = (B // blk,)
    return pl.pallas_call(
        _mlp_body,
        grid=grid,
        in_specs=[
            pl.BlockSpec((blk, 128), lambda i: (i, 0)),
            pl.BlockSpec((blk, 1), lambda i: (i, 0)),
            pl.BlockSpec((H, D), lambda i: (0, 0)),
            pl.BlockSpec((H, 1), lambda i: (0, 0)),
            pl.BlockSpec((D, H), lambda i: (0, 0)),
            pl.BlockSpec((D, 1), lambda i: (0, 0)),
        ],
        out_specs=pl.BlockSpec((D, blk), lambda i: (0, i)),
        out_shape=jax.ShapeDtypeStruct((D, B), jnp.float32),
    )


def kernel(user_ids, table, W1, b1, W2, b2):
    V, D = table.shape
    H = W1.shape[0]
    B = user_ids.shape[0]
    info = plsc.get_sparse_core_info()
    NW = info.num_cores * info.num_subcores
    ids = user_ids.astype(jnp.int32)
    table2 = table.reshape(V // 2, 2 * D)
    idx2 = ids // 2
    par = (ids % 2).astype(jnp.float32).reshape(B, 1)
    x2 = _make_gather(V // 2, B)(table2, idx2).reshape(B, 128)
    mlp = _make_mlp(B, D, H, 2048)
    out_t = mlp(x2, par, W1, b1.reshape(H, 1), W2, b2.reshape(D, 1))
    return out_t.T


# speedup vs baseline: 1.0245x; 1.0245x over previous
import functools

import jax
import jax.numpy as jnp
from jax import lax
from jax.experimental import pallas as pl
from jax.experimental.pallas import tpu as pltpu
from jax.experimental.pallas import tpu_sc as plsc

_IDX_CHUNK = 128


@functools.lru_cache(maxsize=None)
def _make_gather(V2, B):
    info = plsc.get_sparse_core_info()
    NC, NS = info.num_cores, info.num_subcores
    NW = NC * NS
    b_per_w = B // NW
    n_chunks = b_per_w // _IDX_CHUNK
    mesh = plsc.VectorSubcoreMesh(core_axis_name="c", subcore_axis_name="s")

    @functools.partial(
        pl.kernel,
        mesh=mesh,
        out_type=jax.ShapeDtypeStruct((NW, b_per_w, 128), jnp.float32),
        scratch_types=[
            pltpu.VMEM((n_chunks, _IDX_CHUNK), jnp.int32),
            pltpu.VMEM((b_per_w, 128), jnp.float32),
            pltpu.SemaphoreType.DMA,
        ],
        compiler_params=pltpu.CompilerParams(
            use_tc_tiling_on_sc=True, needs_layout_passes=False
        ),
    )
    def gather_k(table_hbm, idx_hbm, out_hbm, idx_v, rows_v, sem):
        wid = lax.axis_index("s") * NC + lax.axis_index("c")
        base = wid * b_per_w
        for j in range(n_chunks):
            pltpu.sync_copy(
                idx_hbm.at[pl.ds(base + j * _IDX_CHUNK, _IDX_CHUNK)],
                idx_v.at[j],
            )
        copies = []
        for j in range(n_chunks):
            copies.append(
                pltpu.async_copy(
                    table_hbm.at[idx_v.at[j]],
                    rows_v.at[pl.ds(j * _IDX_CHUNK, _IDX_CHUNK)],
                    sem,
                )
            )
        for c in copies:
            c.wait()
        pltpu.sync_copy(rows_v, out_hbm.at[wid])

    return gather_k


def _mlp_body(x_ref, par_ref, w1_ref, b1_ref, w2_ref, b2_ref, o_ref):
    x2 = x_ref[...]
    take_hi = par_ref[...] > 0.5
    x = jnp.where(take_hi, x2[:, 64:], x2[:, :64])
    h_t = lax.dot_general(
        w1_ref[...], x, (((1,), (1,)), ((), ())),
        preferred_element_type=jnp.float32,
    )
    h_t = jnp.maximum(h_t + b1_ref[...], 0.0)
    y_t = lax.dot_general(
        w2_ref[...], h_t, (((1,), (0,)), ((), ())),
        preferred_element_type=jnp.float32,
    )
    y_t = y_t + b2_ref[...]
    norm = jnp.sqrt(jnp.sum(y_t * y_t, axis=0, keepdims=True))
    o_ref[...] = y_t / jnp.maximum(norm, 1e-12)


@functools.lru_cache(maxsize=None)
def _make_mlp(B, D, H, blk):
    grid = (B // blk,)
    return pl.pallas_call(
        _mlp_body,
        grid=grid,
        in_specs=[
            pl.BlockSpec((blk, 128), lambda i: (i, 0)),
            pl.BlockSpec((blk, 1), lambda i: (i, 0)),
            pl.BlockSpec((H, D), lambda i: (0, 0)),
            pl.BlockSpec((H, 1), lambda i: (0, 0)),
            pl.BlockSpec((D, H), lambda i: (0, 0)),
            pl.BlockSpec((D, 1), lambda i: (0, 0)),
        ],
        out_specs=pl.BlockSpec((D, blk), lambda i: (0, i)),
        out_shape=jax.ShapeDtypeStruct((D, B), jnp.float32),
    )


def kernel(user_ids, table, W1, b1, W2, b2):
    V, D = table.shape
    H = W1.shape[0]
    B = user_ids.shape[0]
    info = plsc.get_sparse_core_info()
    NW = info.num_cores * info.num_subcores
    ids = user_ids.astype(jnp.int32)
    table2 = table.reshape(V // 2, 2 * D)
    idx2 = ids // 2
    par = (ids % 2).astype(jnp.float32).reshape(B, 1)
    x2 = _make_gather(V // 2, B)(table2, idx2).reshape(B, 128)
    mlp = _make_mlp(B, D, H, 4096)
    out_t = mlp(x2, par, W1, b1.reshape(H, 1), W2, b2.reshape(D, 1))
    return out_t.T
